# baseline (speedup 1.0000x reference)
import jax
import jax.numpy as jnp
from jax.experimental import pallas as pl
from jax.experimental.pallas import tpu as pltpu

_E = 32
_H = 64
_PACK = 4
_ROWS = 16384
_PROWS = _ROWS // _PACK
_LANES = _PACK * _E
_HB = _PACK * _H
_NCHUNK = 4
_CROWS = _PROWS // _NCHUNK


def _expand(w_half):
    tiled = jnp.tile(w_half, (_PACK, _PACK))
    r = jax.lax.broadcasted_iota(jnp.int32, (_LANES, _HB), 0)
    c = jax.lax.broadcasted_iota(jnp.int32, (_LANES, _HB), 1)
    return jnp.where((r // _E) == (c // _H), tiled, 0)


def _chunk_copy(x_hbm, xv, sems, h, c):
    return pltpu.make_async_copy(
        x_hbm.at[h, pl.ds(c * _CROWS, _CROWS), :],
        xv.at[h, pl.ds(c * _CROWS, _CROWS), :],
        sems.at[h, c],
    )


def _mlp_body(x_hbm, w0_ref, b0_ref, w1_ref, b1_ref, out_ref, xv, sems):
    for h in range(2):
        for c in range(_NCHUNK):
            _chunk_copy(x_hbm, xv, sems, h, c).start()

    w0 = w0_ref[...].astype(jnp.bfloat16)
    wa = _expand(w0[:_E])
    wb = _expand(w0[_E:])
    b0t = jnp.tile(b0_ref[...], (1, _PACK))
    r = jax.lax.broadcasted_iota(jnp.int32, (_HB, _PACK), 0)
    c2 = jax.lax.broadcasted_iota(jnp.int32, (_HB, _PACK), 1)
    k2 = jnp.where((r // _H) == c2, jnp.tile(w1_ref[...], (_PACK, _PACK)), 0)
    k2 = k2.astype(jnp.bfloat16)

    for c in range(_NCHUNK):
        _chunk_copy(x_hbm, xv, sems, 0, c).wait()
        _chunk_copy(x_hbm, xv, sems, 1, c).wait()
        sl = pl.ds(c * _CROWS, _CROWS)
        hdn = (
            jnp.dot(xv[0, sl, :], wa, preferred_element_type=jnp.float32)
            + jnp.dot(xv[1, sl, :], wb, preferred_element_type=jnp.float32)
            + b0t
        )
        hdn = jnp.maximum(hdn, 0.0).astype(jnp.bfloat16)
        out_ref[sl, :] = (
            jnp.dot(hdn, k2, preferred_element_type=jnp.float32) + b1_ref[...]
        )


def kernel(inputs, W0, b0, W1, b1):
    x = inputs.reshape(2, _PROWS, _LANES).astype(jnp.bfloat16)
    out4 = pl.pallas_call(
        _mlp_body,
        in_specs=[
            pl.BlockSpec(memory_space=pltpu.MemorySpace.HBM),
            pl.BlockSpec(memory_space=pltpu.MemorySpace.VMEM),
            pl.BlockSpec(memory_space=pltpu.MemorySpace.VMEM),
            pl.BlockSpec(memory_space=pltpu.MemorySpace.VMEM),
            pl.BlockSpec(memory_space=pltpu.MemorySpace.VMEM),
        ],
        out_shape=jax.ShapeDtypeStruct((_PROWS, _PACK), jnp.float32),
        scratch_shapes=[
            pltpu.VMEM((2, _PROWS, _LANES), jnp.bfloat16),
            pltpu.SemaphoreType.DMA((2, _NCHUNK)),
        ],
    )(x, W0, b0.reshape(1, _H), W1, b1.reshape(1, 1))
    return out4.reshape(_ROWS, 1)


# speedup vs baseline: 1.0791x; 1.0791x over previous
import jax
import jax.numpy as jnp
from jax.experimental import pallas as pl
from jax.experimental.pallas import tpu as pltpu

_E = 32
_H = 64
_PACK = 4
_ROWS = 16384
_PROWS = _ROWS // _PACK
_LANES = _PACK * _E
_HB = _PACK * _H


def _expand(w_half):
    tiled = jnp.tile(w_half, (_PACK, _PACK))
    r = jax.lax.broadcasted_iota(jnp.int32, (_LANES, _HB), 0)
    c = jax.lax.broadcasted_iota(jnp.int32, (_LANES, _HB), 1)
    return jnp.where((r // _E) == (c // _H), tiled, 0)


def _mlp_body(x_ref, w0_ref, b0_ref, w1_ref, b1_ref, out_ref):
    x = x_ref[...]
    w0 = w0_ref[...].astype(jnp.bfloat16)
    h = (
        jnp.dot(x[0], _expand(w0[:_E]), preferred_element_type=jnp.float32)
        + jnp.dot(x[1], _expand(w0[_E:]), preferred_element_type=jnp.float32)
        + jnp.tile(b0_ref[...], (1, _PACK))
    )
    h = jnp.maximum(h, 0.0).astype(jnp.bfloat16)

    r = jax.lax.broadcasted_iota(jnp.int32, (_HB, _PACK), 0)
    c = jax.lax.broadcasted_iota(jnp.int32, (_HB, _PACK), 1)
    k2 = jnp.where((r // _H) == c, jnp.tile(w1_ref[...], (_PACK, _PACK)), 0)
    k2 = k2.astype(jnp.bfloat16)

    out_ref[...] = (
        jnp.dot(h, k2, preferred_element_type=jnp.float32) + b1_ref[...]
    )


def kernel(inputs, W0, b0, W1, b1):
    x = inputs.reshape(2, _PROWS, _LANES).astype(jnp.bfloat16)
    out4 = pl.pallas_call(
        _mlp_body,
        out_shape=jax.ShapeDtypeStruct((_PROWS, _PACK), jnp.float32),
    )(x, W0, b0.reshape(1, _H), W1, b1.reshape(1, 1))
    return out4.reshape(_ROWS, 1)
